# baseline (speedup 1.0000x reference)
import functools

import jax
import jax.numpy as jnp
from jax import lax
from jax.experimental import pallas as pl
from jax.experimental.pallas import tpu as pltpu
from jax.experimental.pallas import tpu_sc as plsc

B = 16384
L = 200
VOCAB = 22
D = 128
VP = 24

NC = 2
NS = 16
NW = NC * NS
RPW = B // NW
CH = 128
NCHUNK = RPW // CH
SUB = CH // 16
LBLK = 16
NBLK = L // LBLK
LTAIL = L - NBLK * LBLK


def _sc_counts(ctx_hbm, out_hbm, ctx_a, ctx_b, cnt_buf, cnt_buf2, sem_a, sem_b):
    wid = lax.axis_index("s") * NC + lax.axis_index("c")
    lane = lax.iota(jnp.int32, 16)
    ones = jnp.ones((16,), jnp.float32)
    zeros16 = jnp.zeros((16,), jnp.float32)
    rows = [lane + s * 16 for s in range(SUB)]
    phase = (lane * 9) & 15

    def src(chunk):
        return ctx_hbm.at[pl.ds(chunk * CH, CH), :]

    def process(ctx_buf, chunk):
        def zbody(i, carry):
            for k in range(8):
                cnt_buf[pl.ds(i * 128 + k * 16, 16)] = zeros16
                cnt_buf2[pl.ds(i * 128 + k * 16, 16)] = zeros16
            return carry

        lax.fori_loop(0, CH * VP // 128, zbody, 0)

        @plsc.parallel_loop(0, NBLK * LBLK, step=LBLK)
        def _(lblk):
            for lo in range(LBLK):
                pos = lblk + ((lo + phase) & 15)
                dst = cnt_buf if lo % 2 == 0 else cnt_buf2
                for s in range(SUB):
                    v = plsc.load_gather(ctx_buf, [rows[s], pos])
                    plsc.addupdate_scatter(dst, [(v << 7) + rows[s]], ones)

        for lo in range(LTAIL):
            pos = jnp.full((16,), NBLK * LBLK + lo, jnp.int32)
            dst = cnt_buf if lo % 2 == 0 else cnt_buf2
            for s in range(SUB):
                v = plsc.load_gather(ctx_buf, [rows[s], pos])
                plsc.addupdate_scatter(dst, [(v << 7) + rows[s]], ones)

        def mbody(i, carry):
            for k in range(8):
                sl = pl.ds(i * 128 + k * 16, 16)
                cnt_buf[sl] = cnt_buf[sl] + cnt_buf2[sl]
            return carry

        lax.fori_loop(0, CH * VP // 128, mbody, 0)

        pltpu.sync_copy(cnt_buf, out_hbm.at[pl.ds(chunk * (CH * VP), CH * VP)])

    base = wid * NCHUNK
    pltpu.async_copy(src(base), ctx_a, sem_a)

    def pair_body(p, carry):
        c0 = base + 2 * p
        pltpu.async_copy(src(c0 + 1), ctx_b, sem_b)
        pltpu.make_async_copy(src(base), ctx_a, sem_a).wait()
        process(ctx_a, c0)

        @pl.when(2 * p + 2 < NCHUNK)
        def _():
            pltpu.async_copy(src(c0 + 2), ctx_a, sem_a)

        pltpu.make_async_copy(src(base), ctx_b, sem_b).wait()
        process(ctx_b, c0 + 1)
        return carry

    lax.fori_loop(0, NCHUNK // 2, pair_body, 0)


@functools.cache
def _sc_counts_fn():
    return functools.partial(
        pl.kernel,
        out_type=jax.ShapeDtypeStruct((B * VP,), jnp.float32),
        mesh=plsc.VectorSubcoreMesh(
            core_axis_name="c", subcore_axis_name="s", num_cores=NC, num_subcores=NS
        ),
        scratch_types=[
            pltpu.VMEM((CH, L), jnp.int32),
            pltpu.VMEM((CH, L), jnp.int32),
            pltpu.VMEM((CH * VP,), jnp.float32),
            pltpu.VMEM((CH * VP,), jnp.float32),
            pltpu.SemaphoreType.DMA,
            pltpu.SemaphoreType.DMA,
        ],
        compiler_params=pltpu.CompilerParams(
            needs_layout_passes=False, use_tc_tiling_on_sc=True
        ),
    )(_sc_counts)


NB = 2048
CPB = NB // CH


def _tc_body(cnt_ref, word_ref, tab_ref, w_ref, b_ref, out_ref, t1_ref, t2_ref):
    @pl.when(pl.program_id(0) == 0)
    def _():
        t1_ref[...] = jnp.dot(
            tab_ref[...], w_ref[0:D, :], preferred_element_type=jnp.float32
        )
        t2_ref[...] = jnp.dot(
            tab_ref[...], w_ref[D:, :], preferred_element_type=jnp.float32
        ) * (1.0 / L)

    woh = (word_ref[...] == lax.broadcasted_iota(jnp.int32, (NB, VP), 1)).astype(
        jnp.float32
    )
    ctx_part = lax.dot_general(
        cnt_ref[...],
        t2_ref[...],
        dimension_numbers=(((1,), (0,)), ((), ())),
        preferred_element_type=jnp.float32,
    ).reshape(NB, D)
    out_ref[...] = (
        jnp.dot(woh, t1_ref[...], preferred_element_type=jnp.float32)
        + ctx_part
        + b_ref[...]
    )


def _tc_call(counts3, word2, tablep, W, b2):
    return pl.pallas_call(
        _tc_body,
        grid=(B // NB,),
        in_specs=[
            pl.BlockSpec((CPB, VP, CH), lambda i: (i, 0, 0)),
            pl.BlockSpec((NB, 1), lambda i: (i, 0)),
            pl.BlockSpec((VP, D), lambda i: (0, 0)),
            pl.BlockSpec((2 * D, D), lambda i: (0, 0)),
            pl.BlockSpec((1, D), lambda i: (0, 0)),
        ],
        out_specs=pl.BlockSpec((NB, D), lambda i: (i, 0)),
        out_shape=jax.ShapeDtypeStruct((B, D), jnp.float32),
        scratch_shapes=[
            pltpu.VMEM((VP, D), jnp.float32),
            pltpu.VMEM((VP, D), jnp.float32),
        ],
    )(counts3, word2, tablep, W, b2)


def kernel(word_idx, context_idx, table, W, b):
    counts = _sc_counts_fn()(context_idx)
    tablep = jnp.concatenate(
        [table, jnp.zeros((VP - VOCAB, D), jnp.float32)], axis=0
    )
    return _tc_call(
        counts.reshape(B // CH, VP, CH),
        word_idx.reshape(B, 1),
        tablep,
        W,
        b.reshape(1, D),
    )


# speedup vs baseline: 1.0149x; 1.0149x over previous
import functools

import jax
import jax.numpy as jnp
from jax import lax
from jax.experimental import pallas as pl
from jax.experimental.pallas import tpu as pltpu
from jax.experimental.pallas import tpu_sc as plsc

B = 16384
L = 200
VOCAB = 22
D = 128
VP = 24

NC = 2
NS = 16
NW = NC * NS
RPW = B // NW
CH = 128
NCHUNK = RPW // CH
SUB = CH // 16
LBLK = 16
NBLK = L // LBLK
LTAIL = L - NBLK * LBLK


def _sc_counts(ctx_hbm, out_hbm, ctx_a, ctx_b, cnt_buf, sem_a, sem_b):
    wid = lax.axis_index("s") * NC + lax.axis_index("c")
    lane = lax.iota(jnp.int32, 16)
    ones = jnp.ones((16,), jnp.float32)
    zeros16 = jnp.zeros((16,), jnp.float32)
    rows = [lane + s * 16 for s in range(SUB)]
    phase = (lane * 9) & 15

    def src(chunk):
        return ctx_hbm.at[pl.ds(chunk * CH, CH), :]

    def process(ctx_buf, chunk):
        def zbody(i, carry):
            for k in range(8):
                cnt_buf[pl.ds(i * 128 + k * 16, 16)] = zeros16
            return carry

        lax.fori_loop(0, CH * VP // 128, zbody, 0)

        @plsc.parallel_loop(0, NBLK * LBLK, step=LBLK)
        def _(lblk):
            for lo in range(LBLK):
                pos = lblk + ((lo + phase) & 15)
                for s in range(SUB):
                    v = plsc.load_gather(ctx_buf, [rows[s], pos])
                    plsc.addupdate_scatter(cnt_buf, [(v << 7) + rows[s]], ones)

        for lo in range(LTAIL):
            pos = jnp.full((16,), NBLK * LBLK + lo, jnp.int32)
            for s in range(SUB):
                v = plsc.load_gather(ctx_buf, [rows[s], pos])
                plsc.addupdate_scatter(cnt_buf, [(v << 7) + rows[s]], ones)

        pltpu.sync_copy(cnt_buf, out_hbm.at[pl.ds(chunk * (CH * VP), CH * VP)])

    base = wid * NCHUNK
    pltpu.async_copy(src(base), ctx_a, sem_a)

    def pair_body(p, carry):
        c0 = base + 2 * p
        pltpu.async_copy(src(c0 + 1), ctx_b, sem_b)
        pltpu.make_async_copy(src(base), ctx_a, sem_a).wait()
        process(ctx_a, c0)

        @pl.when(2 * p + 2 < NCHUNK)
        def _():
            pltpu.async_copy(src(c0 + 2), ctx_a, sem_a)

        pltpu.make_async_copy(src(base), ctx_b, sem_b).wait()
        process(ctx_b, c0 + 1)
        return carry

    lax.fori_loop(0, NCHUNK // 2, pair_body, 0)


@functools.cache
def _sc_counts_fn():
    return functools.partial(
        pl.kernel,
        out_type=jax.ShapeDtypeStruct((B * VP,), jnp.float32),
        mesh=plsc.VectorSubcoreMesh(
            core_axis_name="c", subcore_axis_name="s", num_cores=NC, num_subcores=NS
        ),
        scratch_types=[
            pltpu.VMEM((CH, L), jnp.int32),
            pltpu.VMEM((CH, L), jnp.int32),
            pltpu.VMEM((CH * VP,), jnp.float32),
            pltpu.SemaphoreType.DMA,
            pltpu.SemaphoreType.DMA,
        ],
        compiler_params=pltpu.CompilerParams(
            needs_layout_passes=False, use_tc_tiling_on_sc=True
        ),
    )(_sc_counts)


NB = 2048
CPB = NB // CH


def _tc_body(cnt_ref, word_ref, tab_ref, w_ref, b_ref, out_ref, t1_ref, t2_ref):
    @pl.when(pl.program_id(0) == 0)
    def _():
        t1_ref[...] = jnp.dot(
            tab_ref[...], w_ref[0:D, :], preferred_element_type=jnp.float32
        )
        t2_ref[...] = jnp.dot(
            tab_ref[...], w_ref[D:, :], preferred_element_type=jnp.float32
        ) * (1.0 / L)

    woh = (word_ref[...] == lax.broadcasted_iota(jnp.int32, (NB, VP), 1)).astype(
        jnp.float32
    )
    ctx_part = lax.dot_general(
        cnt_ref[...],
        t2_ref[...],
        dimension_numbers=(((1,), (0,)), ((), ())),
        preferred_element_type=jnp.float32,
    ).reshape(NB, D)
    out_ref[...] = (
        jnp.dot(woh, t1_ref[...], preferred_element_type=jnp.float32)
        + ctx_part
        + b_ref[...]
    )


def _tc_call(counts3, word2, tablep, W, b2):
    return pl.pallas_call(
        _tc_body,
        grid=(B // NB,),
        in_specs=[
            pl.BlockSpec((CPB, VP, CH), lambda i: (i, 0, 0)),
            pl.BlockSpec((NB, 1), lambda i: (i, 0)),
            pl.BlockSpec((VP, D), lambda i: (0, 0)),
            pl.BlockSpec((2 * D, D), lambda i: (0, 0)),
            pl.BlockSpec((1, D), lambda i: (0, 0)),
        ],
        out_specs=pl.BlockSpec((NB, D), lambda i: (i, 0)),
        out_shape=jax.ShapeDtypeStruct((B, D), jnp.float32),
        scratch_shapes=[
            pltpu.VMEM((VP, D), jnp.float32),
            pltpu.VMEM((VP, D), jnp.float32),
        ],
    )(counts3, word2, tablep, W, b2)


def kernel(word_idx, context_idx, table, W, b):
    counts = _sc_counts_fn()(context_idx)
    tablep = jnp.concatenate(
        [table, jnp.zeros((VP - VOCAB, D), jnp.float32)], axis=0
    )
    return _tc_call(
        counts.reshape(B // CH, VP, CH),
        word_idx.reshape(B, 1),
        tablep,
        W,
        b.reshape(1, D),
    )
